# baseline (speedup 1.0000x reference)
import jax
import jax.numpy as jnp
from jax.experimental import pallas as pl
from jax.experimental.pallas import tpu as pltpu

_NVERT = 128
_D = 32
_DIM = 768
_H = 12
_HD = _DIM // _H
_W = 8
_WD = _W * _D
_ROWS = _NVERT * _D
_BM = 256
_CB = _BM // _D
_NT = _ROWS // _BM


_NCOPY = 2 * _CB * _W


def _gather_copies(which_ref, k_s, v_s, kn_s, vn_s, sem, j):
    b = jax.lax.rem(j, 2)
    copies = []
    for c_i in range(_CB):
        n = j * _CB + c_i
        for w in range(_W):
            c = which_ref[n, w]
            dst = pl.ds((c_i * _W + w) * _D, _D)
            src = pl.ds(c * _D, _D)
            copies.append(pltpu.make_async_copy(
                k_s.at[src, :], kn_s.at[b, dst, :], sem.at[b]))
            copies.append(pltpu.make_async_copy(
                v_s.at[src, :], vn_s.at[b, dst, :], sem.at[b]))
    return copies


def _issue_gather(which_ref, k_s, v_s, kn_s, vn_s, sem, j):
    for cp in _gather_copies(which_ref, k_s, v_s, kn_s, vn_s, sem, j):
        cp.start()


def _body(which_ref, x_ref, wqk_ref, wv_ref, bqkv_ref, m_ref, wproj_ref,
          bproj_ref, o_ref, q_s, k_s, v_s, kn_s, vn_s, s_s, p_s, y_s, sem):
    i = pl.program_id(0)

    @pl.when(i < _NT)
    def _qkv():
        rows = pl.ds(i * _BM, _BM)
        x = x_ref[...]
        acc = jnp.dot(x, wqk_ref[...],
                      preferred_element_type=jnp.float32) + bqkv_ref[:, :2 * _DIM]
        accv = jnp.dot(x.astype(jnp.bfloat16), wv_ref[...],
                       preferred_element_type=jnp.float32) + bqkv_ref[:, 2 * _DIM:]
        q_s[rows, :] = acc[:, :_DIM] * jnp.float32(_HD ** 0.5)
        k_s[rows, :] = acc[:, _DIM:2 * _DIM]
        v_s[rows, :] = accv.astype(jnp.bfloat16)

    @pl.when(i == _NT)
    def _first_gather():
        _issue_gather(which_ref, k_s, v_s, kn_s, vn_s, sem, 0)

    @pl.when(i >= _NT)
    def _attn():
        j = i - _NT
        b = jax.lax.rem(j, 2)
        qrows = pl.ds(j * _BM, _BM)
        @pl.when(j < _NT - 1)
        def _next_gather():
            _issue_gather(which_ref, k_s, v_s, kn_s, vn_s, sem, j + 1)

        for cp in _gather_copies(which_ref, k_s, v_s, kn_s, vn_s, sem, j):
            cp.wait()
        q = q_s[qrows, :]
        for c_i in range(_CB):
            rs = slice(c_i * _D, (c_i + 1) * _D)
            krs = slice(c_i * _WD, (c_i + 1) * _WD)
            madd = m_ref[c_i]
            for h in range(_H):
                sl = slice(h * _HD, (h + 1) * _HD)
                s = jax.lax.dot_general(q[rs, sl], kn_s[b, krs, sl],
                                        (((1,), (1,)), ((), ())),
                                        preferred_element_type=jnp.float32)
                s_s[rs, h * _WD:(h + 1) * _WD] = s + madd
        recips = []
        for h in range(_H):
            cs = slice(h * _WD, (h + 1) * _WD)
            s = s_s[:, cs]
            mx = jnp.max(s, axis=-1, keepdims=True)
            p = jnp.exp(s - mx)
            denom = jnp.sum(p, axis=-1, keepdims=True)
            recips.append(1.0 / denom)
            p_s[:, cs] = p.astype(jnp.bfloat16)
        for c_i in range(_CB):
            rs = slice(c_i * _D, (c_i + 1) * _D)
            krs = slice(c_i * _WD, (c_i + 1) * _WD)
            for h in range(_H):
                sl = slice(h * _HD, (h + 1) * _HD)
                p = p_s[rs, h * _WD:(h + 1) * _WD]
                yh = jax.lax.dot_general(
                    p, vn_s[b, krs, sl], (((1,), (0,)), ((), ())),
                    preferred_element_type=jnp.float32)
                y_s[rs, sl] = (yh * recips[h][rs]).astype(jnp.bfloat16)
        o_ref[...] = jnp.dot(y_s[...], wproj_ref[...],
                             preferred_element_type=jnp.float32) + bproj_ref[...]


def kernel(x, W_qkv, b_qkv, W_proj, b_proj, which, mask):
    xm = x.reshape(_ROWS, _DIM)
    madd = jnp.where(mask, 0.0, -1e30).astype(jnp.float32)
    madd = madd.reshape(_NVERT, 1, _WD)
    wproj_bf = W_proj.astype(jnp.bfloat16)

    out = pl.pallas_call(
        _body,
        grid=(2 * _NT,),
        in_specs=[
            pl.BlockSpec(memory_space=pltpu.SMEM),
            pl.BlockSpec((_BM, _DIM), lambda i: (jnp.minimum(i, _NT - 1), 0)),
            pl.BlockSpec((_DIM, 2 * _DIM), lambda i: (0, 0)),
            pl.BlockSpec((_DIM, _DIM), lambda i: (0, 0)),
            pl.BlockSpec((1, 3 * _DIM), lambda i: (0, 0)),
            pl.BlockSpec((_CB, 1, _WD),
                         lambda i: (jnp.maximum(i - _NT, 0), 0, 0)),
            pl.BlockSpec((_DIM, _DIM), lambda i: (0, 0)),
            pl.BlockSpec((1, _DIM), lambda i: (0, 0)),
        ],
        out_specs=pl.BlockSpec((_BM, _DIM), lambda i: (jnp.maximum(i - _NT, 0), 0)),
        out_shape=jax.ShapeDtypeStruct((_ROWS, _DIM), jnp.float32),
        scratch_shapes=[
            pltpu.VMEM((_ROWS, _DIM), jnp.float32),
            pltpu.VMEM((_ROWS, _DIM), jnp.float32),
            pltpu.VMEM((_ROWS, _DIM), jnp.bfloat16),
            pltpu.VMEM((2, _CB * _WD, _DIM), jnp.float32),
            pltpu.VMEM((2, _CB * _WD, _DIM), jnp.bfloat16),
            pltpu.VMEM((_BM, _H * _WD), jnp.float32),
            pltpu.VMEM((_BM, _H * _WD), jnp.bfloat16),
            pltpu.VMEM((_BM, _DIM), jnp.bfloat16),
            pltpu.SemaphoreType.DMA((2,)),
        ],
        compiler_params=pltpu.CompilerParams(
            vmem_limit_bytes=110 * 1024 * 1024,
        ),
    )(which, xm, W_qkv[:, :2 * _DIM], W_qkv[:, 2 * _DIM:].astype(jnp.bfloat16),
      b_qkv.reshape(1, 3 * _DIM), madd, wproj_bf, b_proj.reshape(1, _DIM))

    return out.reshape(1, _NVERT, _D, _DIM)


# speedup vs baseline: 2.3284x; 2.3284x over previous
import jax
import jax.numpy as jnp
from jax.experimental import pallas as pl
from jax.experimental.pallas import tpu as pltpu

_NVERT = 128
_D = 32
_DIM = 768
_H = 12
_HD = _DIM // _H
_W = 8
_WD = _W * _D
_ROWS = _NVERT * _D
_BM = 256
_CB = _BM // _D
_NT = _ROWS // _BM


def _body(which_ref, x_ref, wqk_ref, wv_ref, bqkv_ref, m_ref, wproj_ref,
          bproj_ref, o_ref, q_s, k_s, v_s, kn_s, vn_s, s_s, p_s, y_s):
    i = pl.program_id(0)

    @pl.when(i < _NT)
    def _qkv():
        rows = pl.ds(i * _BM, _BM)
        x = x_ref[...]
        acc = jnp.dot(x, wqk_ref[...],
                      preferred_element_type=jnp.float32) + bqkv_ref[:, :2 * _DIM]
        accv = jnp.dot(x.astype(jnp.bfloat16), wv_ref[...],
                       preferred_element_type=jnp.float32) + bqkv_ref[:, 2 * _DIM:]
        q_s[rows, :] = acc[:, :_DIM] * jnp.float32(_HD ** 0.5)
        k_s[rows, :] = acc[:, _DIM:2 * _DIM]
        v_s[rows, :] = accv.astype(jnp.bfloat16)

    @pl.when(i >= _NT)
    def _attn():
        j = i - _NT
        n0 = j * _CB
        qrows = pl.ds(j * _BM, _BM)
        for c_i in range(_CB):
            n = n0 + c_i
            for w in range(_W):
                c = which_ref[n, w]
                dst = pl.ds((c_i * _W + w) * _D, _D)
                src = pl.ds(c * _D, _D)
                kn_s[dst, :] = k_s[src, :]
                vn_s[dst, :] = v_s[src, :]
        q = q_s[qrows, :]
        for c_i in range(_CB):
            rs = slice(c_i * _D, (c_i + 1) * _D)
            krs = slice(c_i * _WD, (c_i + 1) * _WD)
            madd = m_ref[c_i]
            for h in range(_H):
                sl = slice(h * _HD, (h + 1) * _HD)
                s = jax.lax.dot_general(q[rs, sl], kn_s[krs, sl],
                                        (((1,), (1,)), ((), ())),
                                        preferred_element_type=jnp.float32)
                s_s[rs, h * _WD:(h + 1) * _WD] = s + madd
        recips = []
        for h in range(_H):
            cs = slice(h * _WD, (h + 1) * _WD)
            s = s_s[:, cs]
            mx = jnp.max(s, axis=-1, keepdims=True)
            p = jnp.exp(s - mx)
            denom = jnp.sum(p, axis=-1, keepdims=True)
            recips.append(1.0 / denom)
            p_s[:, cs] = p.astype(jnp.bfloat16)
        for c_i in range(_CB):
            rs = slice(c_i * _D, (c_i + 1) * _D)
            krs = slice(c_i * _WD, (c_i + 1) * _WD)
            for h in range(_H):
                sl = slice(h * _HD, (h + 1) * _HD)
                p = p_s[rs, h * _WD:(h + 1) * _WD]
                yh = jax.lax.dot_general(
                    p, vn_s[krs, sl], (((1,), (0,)), ((), ())),
                    preferred_element_type=jnp.float32)
                y_s[rs, sl] = (yh * recips[h][rs]).astype(jnp.bfloat16)
        o_ref[...] = jnp.dot(y_s[...], wproj_ref[...],
                             preferred_element_type=jnp.float32) + bproj_ref[...]


def kernel(x, W_qkv, b_qkv, W_proj, b_proj, which, mask):
    xm = x.reshape(_ROWS, _DIM)
    madd = jnp.where(mask, 0.0, -1e30).astype(jnp.float32)
    madd = madd.reshape(_NVERT, 1, _WD)
    wproj_bf = W_proj.astype(jnp.bfloat16)

    out = pl.pallas_call(
        _body,
        grid=(2 * _NT,),
        in_specs=[
            pl.BlockSpec(memory_space=pltpu.SMEM),
            pl.BlockSpec((_BM, _DIM), lambda i: (jnp.minimum(i, _NT - 1), 0)),
            pl.BlockSpec((_DIM, 2 * _DIM), lambda i: (0, 0)),
            pl.BlockSpec((_DIM, _DIM), lambda i: (0, 0)),
            pl.BlockSpec((1, 3 * _DIM), lambda i: (0, 0)),
            pl.BlockSpec((_CB, 1, _WD),
                         lambda i: (jnp.maximum(i - _NT, 0), 0, 0)),
            pl.BlockSpec((_DIM, _DIM), lambda i: (0, 0)),
            pl.BlockSpec((1, _DIM), lambda i: (0, 0)),
        ],
        out_specs=pl.BlockSpec((_BM, _DIM), lambda i: (jnp.maximum(i - _NT, 0), 0)),
        out_shape=jax.ShapeDtypeStruct((_ROWS, _DIM), jnp.float32),
        scratch_shapes=[
            pltpu.VMEM((_ROWS, _DIM), jnp.float32),
            pltpu.VMEM((_ROWS, _DIM), jnp.float32),
            pltpu.VMEM((_ROWS, _DIM), jnp.bfloat16),
            pltpu.VMEM((_CB * _WD, _DIM), jnp.float32),
            pltpu.VMEM((_CB * _WD, _DIM), jnp.bfloat16),
            pltpu.VMEM((_BM, _H * _WD), jnp.float32),
            pltpu.VMEM((_BM, _H * _WD), jnp.bfloat16),
            pltpu.VMEM((_BM, _DIM), jnp.bfloat16),
        ],
        compiler_params=pltpu.CompilerParams(
            vmem_limit_bytes=110 * 1024 * 1024,
        ),
    )(which, xm, W_qkv[:, :2 * _DIM], W_qkv[:, 2 * _DIM:].astype(jnp.bfloat16),
      b_qkv.reshape(1, 3 * _DIM), madd, wproj_bf, b_proj.reshape(1, _DIM))

    return out.reshape(1, _NVERT, _D, _DIM)


# speedup vs baseline: 3.2662x; 1.4028x over previous
import jax
import jax.numpy as jnp
from jax.experimental import pallas as pl
from jax.experimental.pallas import tpu as pltpu

_NVERT = 128
_D = 32
_DIM = 768
_H = 12
_HD = _DIM // _H
_W = 8
_WD = _W * _D
_ROWS = _NVERT * _D
_BM = 256
_CB = _BM // _D
_NT = _ROWS // _BM


def _body(which_ref, x_ref, wqk_ref, wv_ref, bqkv_ref, m_ref, wproj_ref,
          bproj_ref, o_ref, q_s, k_s, v_s, kn_s, vn_s, s_s, p_s, y_s, d_s):
    i = pl.program_id(0)

    @pl.when(i < _NT)
    def _qkv():
        rows = pl.ds(i * _BM, _BM)
        x = x_ref[...]
        acc = jnp.dot(x, wqk_ref[:, :2 * _DIM],
                      preferred_element_type=jnp.float32) + bqkv_ref[:, :2 * _DIM]
        accv = jnp.dot(x.astype(jnp.bfloat16), wv_ref[...],
                       preferred_element_type=jnp.float32) + bqkv_ref[:, 2 * _DIM:]
        q_s[rows, :] = acc[:, :_DIM] * jnp.float32(_HD ** 0.5)
        k_s[rows, :] = acc[:, _DIM:2 * _DIM]
        v_s[rows, :] = accv.astype(jnp.bfloat16)

    @pl.when(i >= _NT)
    def _attn():
        j = i - _NT
        n0 = j * _CB
        qrows = pl.ds(j * _BM, _BM)
        for c_i in range(_CB):
            n = n0 + c_i
            for w in range(_W):
                c = which_ref[n, w]
                dst = pl.ds((c_i * _W + w) * _D, _D)
                src = pl.ds(c * _D, _D)
                kn_s[dst, :] = k_s[src, :]
                vn_s[dst, :] = v_s[src, :]
        q = q_s[qrows, :]
        for c_i in range(_CB):
            rs = slice(c_i * _D, (c_i + 1) * _D)
            krs = slice(c_i * _WD, (c_i + 1) * _WD)
            madd = m_ref[c_i]
            for h in range(_H):
                sl = slice(h * _HD, (h + 1) * _HD)
                s = jax.lax.dot_general(q[rs, sl], kn_s[krs, sl],
                                        (((1,), (1,)), ((), ())),
                                        preferred_element_type=jnp.float32)
                s_s[rs, h * _WD:(h + 1) * _WD] = s + madd
        ones_bd = jnp.ones((_WD, 128), jnp.bfloat16)
        for h in range(_H):
            cs = slice(h * _WD, (h + 1) * _WD)
            s = s_s[:, cs]
            mx = jnp.max(s, axis=-1, keepdims=True)
            p = jnp.exp(s - mx).astype(jnp.bfloat16)
            d_s[:, h * 128:(h + 1) * 128] = jnp.dot(
                p, ones_bd, preferred_element_type=jnp.float32)
            p_s[:, cs] = p
        for c_i in range(_CB):
            rs = slice(c_i * _D, (c_i + 1) * _D)
            krs = slice(c_i * _WD, (c_i + 1) * _WD)
            for h in range(_H):
                sl = slice(h * _HD, (h + 1) * _HD)
                p = p_s[rs, h * _WD:(h + 1) * _WD]
                yh = jax.lax.dot_general(
                    p, vn_s[krs, sl], (((1,), (0,)), ((), ())),
                    preferred_element_type=jnp.float32)
                d = d_s[rs, h * 128:h * 128 + _HD]
                y_s[rs, sl] = (yh / d).astype(jnp.bfloat16)
        o_ref[...] = jnp.dot(y_s[...], wproj_ref[...],
                             preferred_element_type=jnp.float32) + bproj_ref[...]


def kernel(x, W_qkv, b_qkv, W_proj, b_proj, which, mask):
    xm = x.reshape(_ROWS, _DIM)
    madd = jnp.where(mask, 0.0, -1e30).astype(jnp.float32)
    madd = madd.reshape(_NVERT, 1, _WD)
    wproj_bf = W_proj.astype(jnp.bfloat16)

    out = pl.pallas_call(
        _body,
        grid=(2 * _NT,),
        in_specs=[
            pl.BlockSpec(memory_space=pltpu.SMEM),
            pl.BlockSpec((_BM, _DIM), lambda i: (jnp.minimum(i, _NT - 1), 0)),
            pl.BlockSpec((_DIM, 3 * _DIM), lambda i: (0, 0)),
            pl.BlockSpec((_DIM, _DIM), lambda i: (0, 0)),
            pl.BlockSpec((1, 3 * _DIM), lambda i: (0, 0)),
            pl.BlockSpec((_CB, 1, _WD),
                         lambda i: (jnp.maximum(i - _NT, 0), 0, 0)),
            pl.BlockSpec((_DIM, _DIM), lambda i: (0, 0)),
            pl.BlockSpec((1, _DIM), lambda i: (0, 0)),
        ],
        out_specs=pl.BlockSpec((_BM, _DIM), lambda i: (jnp.maximum(i - _NT, 0), 0)),
        out_shape=jax.ShapeDtypeStruct((_ROWS, _DIM), jnp.float32),
        scratch_shapes=[
            pltpu.VMEM((_ROWS, _DIM), jnp.float32),
            pltpu.VMEM((_ROWS, _DIM), jnp.float32),
            pltpu.VMEM((_ROWS, _DIM), jnp.bfloat16),
            pltpu.VMEM((_CB * _WD, _DIM), jnp.float32),
            pltpu.VMEM((_CB * _WD, _DIM), jnp.bfloat16),
            pltpu.VMEM((_BM, _H * _WD), jnp.float32),
            pltpu.VMEM((_BM, _H * _WD), jnp.bfloat16),
            pltpu.VMEM((_BM, _DIM), jnp.bfloat16),
            pltpu.VMEM((_BM, _H * 128), jnp.float32),
        ],
        compiler_params=pltpu.CompilerParams(
            vmem_limit_bytes=110 * 1024 * 1024,
        ),
    )(which, xm, W_qkv, W_qkv[:, 2 * _DIM:].astype(jnp.bfloat16),
      b_qkv.reshape(1, 3 * _DIM), madd, wproj_bf, b_proj.reshape(1, _DIM))

    return out.reshape(1, _NVERT, _D, _DIM)


# speedup vs baseline: 3.2737x; 1.0023x over previous
import jax
import jax.numpy as jnp
from jax.experimental import pallas as pl
from jax.experimental.pallas import tpu as pltpu

_NVERT = 128
_D = 32
_DIM = 768
_H = 12
_HD = _DIM // _H
_W = 8
_WD = _W * _D
_ROWS = _NVERT * _D
_BM = 256
_CB = _BM // _D
_NT = _ROWS // _BM


def _body(which_ref, x_ref, wqk_ref, wv_ref, bqkv_ref, m_ref, wproj_ref,
          bproj_ref, o_ref, q_s, k_s, v_s, kn_s, vn_s, s_s, p_s, y_s, d_s):
    i = pl.program_id(0)

    @pl.when(i < _NT)
    def _qkv():
        rows = pl.ds(i * _BM, _BM)
        x = x_ref[...]
        acc = jnp.dot(x, wqk_ref[:, :2 * _DIM],
                      preferred_element_type=jnp.float32) + bqkv_ref[:, :2 * _DIM]
        accv = jnp.dot(x.astype(jnp.bfloat16), wv_ref[...],
                       preferred_element_type=jnp.float32) + bqkv_ref[:, 2 * _DIM:]
        q_s[rows, :] = acc[:, :_DIM] * jnp.float32(_HD ** 0.5)
        k_s[rows, :] = acc[:, _DIM:2 * _DIM]
        v_s[rows, :] = accv.astype(jnp.bfloat16)

    @pl.when(i >= _NT)
    def _attn():
        j = i - _NT
        n0 = j * _CB
        qrows = pl.ds(j * _BM, _BM)
        for c_i in range(_CB):
            n = n0 + c_i
            for w in range(_W):
                c = which_ref[n, w]
                dst = pl.ds((c_i * _W + w) * _D, _D)
                src = pl.ds(c * _D, _D)
                kn_s[dst, :] = k_s[src, :]
                vn_s[dst, :] = v_s[src, :]
        q = q_s[qrows, :]
        for c_i in range(_CB):
            rs = slice(c_i * _D, (c_i + 1) * _D)
            krs = slice(c_i * _WD, (c_i + 1) * _WD)
            madd = m_ref[c_i]
            for h in range(_H):
                sl = slice(h * _HD, (h + 1) * _HD)
                s = jax.lax.dot_general(q[rs, sl], kn_s[krs, sl],
                                        (((1,), (1,)), ((), ())),
                                        preferred_element_type=jnp.float32)
                s_s[rs, h * _WD:(h + 1) * _WD] = s + madd
        ones_bd = jnp.ones((_WD, 128), jnp.bfloat16)
        for h in range(_H):
            cs = slice(h * _WD, (h + 1) * _WD)
            s = s_s[:, cs]
            mx = jnp.max(s, axis=-1, keepdims=True)
            p = jnp.exp(s - mx).astype(jnp.bfloat16)
            d_s[:, h * 128:(h + 1) * 128] = 1.0 / jnp.dot(
                p, ones_bd, preferred_element_type=jnp.float32)
            p_s[:, cs] = p
        for c_i in range(_CB):
            rs = slice(c_i * _D, (c_i + 1) * _D)
            krs = slice(c_i * _WD, (c_i + 1) * _WD)
            for h in range(_H):
                sl = slice(h * _HD, (h + 1) * _HD)
                p = p_s[rs, h * _WD:(h + 1) * _WD]
                yh = jax.lax.dot_general(
                    p, vn_s[krs, sl], (((1,), (0,)), ((), ())),
                    preferred_element_type=jnp.float32)
                d = d_s[rs, h * 128:h * 128 + _HD]
                y_s[rs, sl] = (yh * d).astype(jnp.bfloat16)
        o_ref[...] = jnp.dot(y_s[...], wproj_ref[...],
                             preferred_element_type=jnp.float32) + bproj_ref[...]


def kernel(x, W_qkv, b_qkv, W_proj, b_proj, which, mask):
    xm = x.reshape(_ROWS, _DIM)
    madd = jnp.where(mask, 0.0, -1e30).astype(jnp.float32)
    madd = madd.reshape(_NVERT, 1, _WD)
    wproj_bf = W_proj.astype(jnp.bfloat16)

    out = pl.pallas_call(
        _body,
        grid=(2 * _NT,),
        in_specs=[
            pl.BlockSpec(memory_space=pltpu.SMEM),
            pl.BlockSpec((_BM, _DIM), lambda i: (jnp.minimum(i, _NT - 1), 0)),
            pl.BlockSpec((_DIM, 3 * _DIM), lambda i: (0, 0)),
            pl.BlockSpec((_DIM, _DIM), lambda i: (0, 0)),
            pl.BlockSpec((1, 3 * _DIM), lambda i: (0, 0)),
            pl.BlockSpec((_CB, 1, _WD),
                         lambda i: (jnp.maximum(i - _NT, 0), 0, 0)),
            pl.BlockSpec((_DIM, _DIM), lambda i: (0, 0)),
            pl.BlockSpec((1, _DIM), lambda i: (0, 0)),
        ],
        out_specs=pl.BlockSpec((_BM, _DIM), lambda i: (jnp.maximum(i - _NT, 0), 0)),
        out_shape=jax.ShapeDtypeStruct((_ROWS, _DIM), jnp.float32),
        scratch_shapes=[
            pltpu.VMEM((_ROWS, _DIM), jnp.float32),
            pltpu.VMEM((_ROWS, _DIM), jnp.float32),
            pltpu.VMEM((_ROWS, _DIM), jnp.bfloat16),
            pltpu.VMEM((_CB * _WD, _DIM), jnp.float32),
            pltpu.VMEM((_CB * _WD, _DIM), jnp.bfloat16),
            pltpu.VMEM((_BM, _H * _WD), jnp.float32),
            pltpu.VMEM((_BM, _H * _WD), jnp.bfloat16),
            pltpu.VMEM((_BM, _DIM), jnp.bfloat16),
            pltpu.VMEM((_BM, _H * 128), jnp.float32),
        ],
        compiler_params=pltpu.CompilerParams(
            vmem_limit_bytes=110 * 1024 * 1024,
        ),
    )(which, xm, W_qkv, W_qkv[:, 2 * _DIM:].astype(jnp.bfloat16),
      b_qkv.reshape(1, 3 * _DIM), madd, wproj_bf, b_proj.reshape(1, _DIM))

    return out.reshape(1, _NVERT, _D, _DIM)


# speedup vs baseline: 3.4797x; 1.0629x over previous
import jax
import jax.numpy as jnp
from jax.experimental import pallas as pl
from jax.experimental.pallas import tpu as pltpu

_NVERT = 128
_D = 32
_DIM = 768
_H = 12
_HD = _DIM // _H
_W = 8
_WD = _W * _D
_ROWS = _NVERT * _D
_BM = 512
_CB = _BM // _D
_NT = _ROWS // _BM


def _body(which_ref, x_ref, wqk_ref, wv_ref, bqkv_ref, m_ref, wproj_ref,
          bproj_ref, o_ref, q_s, k_s, v_s, kn_s, vn_s, s_s, p_s, y_s, d_s):
    i = pl.program_id(0)

    @pl.when(i < _NT)
    def _qkv():
        rows = pl.ds(i * _BM, _BM)
        x = x_ref[...]
        acc = jnp.dot(x, wqk_ref[:, :2 * _DIM],
                      preferred_element_type=jnp.float32) + bqkv_ref[:, :2 * _DIM]
        accv = jnp.dot(x.astype(jnp.bfloat16), wv_ref[...],
                       preferred_element_type=jnp.float32) + bqkv_ref[:, 2 * _DIM:]
        q_s[rows, :] = acc[:, :_DIM] * jnp.float32(_HD ** 0.5)
        k_s[rows, :] = acc[:, _DIM:2 * _DIM]
        v_s[rows, :] = accv.astype(jnp.bfloat16)

    @pl.when(i >= _NT)
    def _attn():
        j = i - _NT
        n0 = j * _CB
        qrows = pl.ds(j * _BM, _BM)
        for c_i in range(_CB):
            n = n0 + c_i
            for w in range(_W):
                c = which_ref[n, w]
                dst = pl.ds((c_i * _W + w) * _D, _D)
                src = pl.ds(c * _D, _D)
                kn_s[dst, :] = k_s[src, :]
                vn_s[dst, :] = v_s[src, :]
        q = q_s[qrows, :]
        for c_i in range(_CB):
            rs = slice(c_i * _D, (c_i + 1) * _D)
            krs = slice(c_i * _WD, (c_i + 1) * _WD)
            madd = m_ref[c_i]
            for h in range(_H):
                sl = slice(h * _HD, (h + 1) * _HD)
                s = jax.lax.dot_general(q[rs, sl], kn_s[krs, sl],
                                        (((1,), (1,)), ((), ())),
                                        preferred_element_type=jnp.float32)
                s_s[rs, h * _WD:(h + 1) * _WD] = s + madd
        ones_bd = jnp.ones((_WD, 128), jnp.bfloat16)
        for h in range(_H):
            cs = slice(h * _WD, (h + 1) * _WD)
            s = s_s[:, cs]
            mx = jnp.max(s, axis=-1, keepdims=True)
            p = jnp.exp(s - mx).astype(jnp.bfloat16)
            d_s[:, h * 128:(h + 1) * 128] = 1.0 / jnp.dot(
                p, ones_bd, preferred_element_type=jnp.float32)
            p_s[:, cs] = p
        for c_i in range(_CB):
            rs = slice(c_i * _D, (c_i + 1) * _D)
            krs = slice(c_i * _WD, (c_i + 1) * _WD)
            for h in range(_H):
                sl = slice(h * _HD, (h + 1) * _HD)
                p = p_s[rs, h * _WD:(h + 1) * _WD]
                yh = jax.lax.dot_general(
                    p, vn_s[krs, sl], (((1,), (0,)), ((), ())),
                    preferred_element_type=jnp.float32)
                d = d_s[rs, h * 128:h * 128 + _HD]
                y_s[rs, sl] = (yh * d).astype(jnp.bfloat16)
        o_ref[...] = jnp.dot(y_s[...], wproj_ref[...],
                             preferred_element_type=jnp.float32) + bproj_ref[...]


def kernel(x, W_qkv, b_qkv, W_proj, b_proj, which, mask):
    xm = x.reshape(_ROWS, _DIM)
    madd = jnp.where(mask, 0.0, -1e30).astype(jnp.float32)
    madd = madd.reshape(_NVERT, 1, _WD)
    wproj_bf = W_proj.astype(jnp.bfloat16)

    out = pl.pallas_call(
        _body,
        grid=(2 * _NT,),
        in_specs=[
            pl.BlockSpec(memory_space=pltpu.SMEM),
            pl.BlockSpec((_BM, _DIM), lambda i: (jnp.minimum(i, _NT - 1), 0)),
            pl.BlockSpec((_DIM, 3 * _DIM), lambda i: (0, 0)),
            pl.BlockSpec((_DIM, _DIM), lambda i: (0, 0)),
            pl.BlockSpec((1, 3 * _DIM), lambda i: (0, 0)),
            pl.BlockSpec((_CB, 1, _WD),
                         lambda i: (jnp.maximum(i - _NT, 0), 0, 0)),
            pl.BlockSpec((_DIM, _DIM), lambda i: (0, 0)),
            pl.BlockSpec((1, _DIM), lambda i: (0, 0)),
        ],
        out_specs=pl.BlockSpec((_BM, _DIM), lambda i: (jnp.maximum(i - _NT, 0), 0)),
        out_shape=jax.ShapeDtypeStruct((_ROWS, _DIM), jnp.float32),
        scratch_shapes=[
            pltpu.VMEM((_ROWS, _DIM), jnp.float32),
            pltpu.VMEM((_ROWS, _DIM), jnp.float32),
            pltpu.VMEM((_ROWS, _DIM), jnp.bfloat16),
            pltpu.VMEM((_CB * _WD, _DIM), jnp.float32),
            pltpu.VMEM((_CB * _WD, _DIM), jnp.bfloat16),
            pltpu.VMEM((_BM, _H * _WD), jnp.float32),
            pltpu.VMEM((_BM, _H * _WD), jnp.bfloat16),
            pltpu.VMEM((_BM, _DIM), jnp.bfloat16),
            pltpu.VMEM((_BM, _H * 128), jnp.float32),
        ],
        compiler_params=pltpu.CompilerParams(
            vmem_limit_bytes=110 * 1024 * 1024,
        ),
    )(which, xm, W_qkv, W_qkv[:, 2 * _DIM:].astype(jnp.bfloat16),
      b_qkv.reshape(1, 3 * _DIM), madd, wproj_bf, b_proj.reshape(1, _DIM))

    return out.reshape(1, _NVERT, _D, _DIM)
